# baseline (speedup 1.0000x reference)
import functools

import jax
import jax.numpy as jnp
from jax import lax
from jax.experimental import pallas as pl
from jax.experimental.pallas import tpu as pltpu
from jax.experimental.pallas import tpu_sc as plsc

_INFO = plsc.get_sparse_core_info()
_NC = _INFO.num_cores
_NS = _INFO.num_subcores
_NW = _NC * _NS
_L = 16


def _edge_dot_sc(x, src, dst, n_edges, d):
    epw = n_edges // _NW
    k = 400 if epw % 400 == 0 else 16
    assert epw % k == 0 and k % _L == 0
    n_chunks = epw // k
    groups = k // _L

    mesh = plsc.VectorSubcoreMesh(core_axis_name="c", subcore_axis_name="s")

    @functools.partial(
        pl.kernel,
        mesh=mesh,
        out_type=jax.ShapeDtypeStruct((n_edges,), jnp.float32),
        compiler_params=pltpu.CompilerParams(needs_layout_passes=False),
        scratch_types=[
            pltpu.VMEM((k,), jnp.int32),
            pltpu.VMEM((k,), jnp.int32),
            pltpu.VMEM((k, d), jnp.float32),
            pltpu.VMEM((k, d), jnp.float32),
            pltpu.VMEM((k,), jnp.float32),
            pltpu.SemaphoreType.DMA,
            pltpu.SemaphoreType.DMA,
        ],
    )
    def run(x_hbm, src_hbm, dst_hbm, out_hbm,
            src_v, dst_v, a_v, b_v, o_v, sem_a, sem_b):
        wid = lax.axis_index("s") * _NC + lax.axis_index("c")
        base = wid * epw

        def chunk_body(c, carry):
            cb = base + c * k
            pltpu.sync_copy(src_hbm.at[pl.ds(cb, k)], src_v)
            pltpu.sync_copy(dst_hbm.at[pl.ds(cb, k)], dst_v)
            a_cp = pltpu.async_copy(x_hbm.at[src_v], a_v, sem_a)
            b_cp = pltpu.async_copy(x_hbm.at[dst_v], b_v, sem_b)
            a_cp.wait()
            b_cp.wait()

            lanes = lax.iota(jnp.int32, _L)
            unroll = 8

            def group_body(g, carry2):
                eids = g * _L + lanes

                def feat_body(fb, acc):
                    for u in range(unroll):
                        fv = jnp.full((_L,), fb * unroll + u, jnp.int32)
                        av = plsc.load_gather(a_v, [eids, fv])
                        bv = plsc.load_gather(b_v, [eids, fv])
                        acc = acc + av * bv
                    return acc

                acc = lax.fori_loop(0, d // unroll, feat_body,
                                    jnp.zeros((_L,), jnp.float32))
                o_v[pl.ds(g * _L, _L)] = acc
                return carry2

            lax.fori_loop(0, groups, group_body, 0)
            pltpu.sync_copy(o_v, out_hbm.at[pl.ds(cb, k)])
            return carry

        lax.fori_loop(0, n_chunks, chunk_body, 0)

    return run(x, src, dst)


def kernel(x, edge_index):
    n, d = x.shape
    n_edges = edge_index.shape[1]
    ei = edge_index.astype(jnp.int32)
    return _edge_dot_sc(x, ei[0], ei[1], n_edges, d)


# speedup vs baseline: 4.8932x; 4.8932x over previous
import functools

import jax
import jax.numpy as jnp
from jax import lax
from jax.experimental import pallas as pl
from jax.experimental.pallas import tpu as pltpu
from jax.experimental.pallas import tpu_sc as plsc

_INFO = plsc.get_sparse_core_info()
_NC = _INFO.num_cores
_NS = _INFO.num_subcores
_NW = _NC * _NS
_L = 16


def _edge_dot_sc(x, src, dst, n_edges, d):
    epw = n_edges // _NW
    k = 400 if epw % 400 == 0 else 16
    assert epw % k == 0 and k % _L == 0
    n_chunks = epw // k
    groups = k // _L

    mesh = plsc.VectorSubcoreMesh(core_axis_name="c", subcore_axis_name="s")

    @functools.partial(
        pl.kernel,
        mesh=mesh,
        out_type=jax.ShapeDtypeStruct((n_edges,), jnp.float32),
        compiler_params=pltpu.CompilerParams(needs_layout_passes=False),
        scratch_types=[
            pltpu.VMEM((k,), jnp.int32),
            pltpu.VMEM((k,), jnp.int32),
            pltpu.VMEM((k, d), jnp.float32),
            pltpu.VMEM((k, d), jnp.float32),
            pltpu.VMEM((k,), jnp.float32),
            pltpu.SemaphoreType.DMA,
            pltpu.SemaphoreType.DMA,
        ],
    )
    def run(x_hbm, src_hbm, dst_hbm, out_hbm,
            src_v, dst_v, a_v, b_v, o_v, sem_a, sem_b):
        wid = lax.axis_index("s") * _NC + lax.axis_index("c")
        base = wid * epw

        def chunk_body(c, carry):
            cb = base + c * k
            pltpu.sync_copy(src_hbm.at[pl.ds(cb, k)], src_v)
            pltpu.sync_copy(dst_hbm.at[pl.ds(cb, k)], dst_v)
            a_cp = pltpu.async_copy(x_hbm.at[src_v], a_v, sem_a)
            b_cp = pltpu.async_copy(x_hbm.at[dst_v], b_v, sem_b)
            a_cp.wait()
            b_cp.wait()

            lanes = lax.iota(jnp.int32, _L)
            unroll = 8

            def group_body(g, carry2):
                eids = g * _L + lanes

                def feat_body(fb, acc):
                    for u in range(unroll):
                        fv = (lanes + (fb * unroll + u)) & (d - 1)
                        av = plsc.load_gather(a_v, [eids, fv])
                        bv = plsc.load_gather(b_v, [eids, fv])
                        acc = acc + av * bv
                    return acc

                acc = lax.fori_loop(0, d // unroll, feat_body,
                                    jnp.zeros((_L,), jnp.float32))
                o_v[pl.ds(g * _L, _L)] = acc
                return carry2

            lax.fori_loop(0, groups, group_body, 0)
            pltpu.sync_copy(o_v, out_hbm.at[pl.ds(cb, k)])
            return carry

        lax.fori_loop(0, n_chunks, chunk_body, 0)

    return run(x, src, dst)


def kernel(x, edge_index):
    n, d = x.shape
    n_edges = edge_index.shape[1]
    ei = edge_index.astype(jnp.int32)
    return _edge_dot_sc(x, ei[0], ei[1], n_edges, d)


# speedup vs baseline: 5.4351x; 1.1107x over previous
import functools

import jax
import jax.numpy as jnp
from jax import lax
from jax.experimental import pallas as pl
from jax.experimental.pallas import tpu as pltpu
from jax.experimental.pallas import tpu_sc as plsc

_INFO = plsc.get_sparse_core_info()
_NC = _INFO.num_cores
_NS = _INFO.num_subcores
_NW = _NC * _NS
_L = 16


def _edge_dot_sc(x_pk, src, dst, n_edges, d2):
    epw = n_edges // _NW
    k = 400 if epw % 400 == 0 else 16
    assert epw % k == 0 and k % _L == 0
    n_chunks = epw // k
    groups = k // _L

    mesh = plsc.VectorSubcoreMesh(core_axis_name="c", subcore_axis_name="s")

    @functools.partial(
        pl.kernel,
        mesh=mesh,
        out_type=jax.ShapeDtypeStruct((n_edges,), jnp.float32),
        compiler_params=pltpu.CompilerParams(
            needs_layout_passes=False, use_tc_tiling_on_sc=False),
        scratch_types=[
            pltpu.VMEM((k,), jnp.int32),
            pltpu.VMEM((k,), jnp.int32),
            pltpu.VMEM((k, d2), jnp.int32),
            pltpu.VMEM((k, d2), jnp.int32),
            pltpu.VMEM((k,), jnp.float32),
            pltpu.SemaphoreType.DMA,
            pltpu.SemaphoreType.DMA,
        ],
    )
    def run(x_hbm, src_hbm, dst_hbm, out_hbm,
            src_v, dst_v, a_v, b_v, o_v, sem_a, sem_b):
        wid = lax.axis_index("s") * _NC + lax.axis_index("c")
        base = wid * epw

        def chunk_body(c, carry):
            cb = base + c * k
            pltpu.sync_copy(src_hbm.at[pl.ds(cb, k)], src_v)
            pltpu.sync_copy(dst_hbm.at[pl.ds(cb, k)], dst_v)
            a_cp = pltpu.async_copy(x_hbm.at[src_v], a_v, sem_a)
            b_cp = pltpu.async_copy(x_hbm.at[dst_v], b_v, sem_b)
            a_cp.wait()
            b_cp.wait()

            lanes = lax.iota(jnp.int32, _L)
            unroll = 8

            def group_body(g, carry2):
                eids = g * _L + lanes

                def col_body(cb2, acc):
                    for u in range(unroll):
                        fv = (lanes + (cb2 * unroll + u)) & (d2 - 1)
                        apk = plsc.load_gather(a_v, [eids, fv])
                        bpk = plsc.load_gather(b_v, [eids, fv])
                        ab = plsc.bitcast(apk, jnp.bfloat16)
                        bb = plsc.bitcast(bpk, jnp.bfloat16)
                        plo, phi = plsc.unpack(
                            ab * bb, format=plsc.PackFormat.INTERLEAVED)
                        acc = acc + plo + phi
                    return acc

                acc = lax.fori_loop(0, d2 // unroll, col_body,
                                    jnp.zeros((_L,), jnp.float32))
                o_v[pl.ds(g * _L, _L)] = acc
                return carry2

            lax.fori_loop(0, groups, group_body, 0)
            pltpu.sync_copy(o_v, out_hbm.at[pl.ds(cb, k)])
            return carry

        lax.fori_loop(0, n_chunks, chunk_body, 0)

    return run(x_pk, src, dst)


def kernel(x, edge_index):
    n, d = x.shape
    n_edges = edge_index.shape[1]
    ei = edge_index.astype(jnp.int32)
    x_pk = lax.bitcast_convert_type(
        x.astype(jnp.bfloat16).reshape(n, d // 2, 2), jnp.int32)
    return _edge_dot_sc(x_pk, ei[0], ei[1], n_edges, d // 2)


# speedup vs baseline: 8.2963x; 1.5264x over previous
import functools

import jax
import jax.numpy as jnp
from jax import lax
from jax.experimental import pallas as pl
from jax.experimental.pallas import tpu as pltpu
from jax.experimental.pallas import tpu_sc as plsc

_INFO = plsc.get_sparse_core_info()
_NC = _INFO.num_cores
_NS = _INFO.num_subcores
_NW = _NC * _NS
_L = 16


def _edge_dot_sc(x_pk, src, dst, n_edges, d2):
    epw = n_edges // _NW
    k = 400 if epw % 400 == 0 else 16
    assert epw % k == 0 and k % _L == 0
    n_chunks = epw // k
    groups = k // _L

    mesh = plsc.VectorSubcoreMesh(core_axis_name="c", subcore_axis_name="s")

    @functools.partial(
        pl.kernel,
        mesh=mesh,
        out_type=jax.ShapeDtypeStruct((n_edges,), jnp.float32),
        compiler_params=pltpu.CompilerParams(
            needs_layout_passes=False, use_tc_tiling_on_sc=False),
        scratch_types=[
            [pltpu.VMEM((k,), jnp.int32)] * 2,
            [pltpu.VMEM((k,), jnp.int32)] * 2,
            [pltpu.VMEM((k, d2), jnp.int32)] * 2,
            [pltpu.VMEM((k, d2), jnp.int32)] * 2,
            [pltpu.VMEM((k,), jnp.float32)] * 2,
            [pltpu.SemaphoreType.DMA] * 2,
            [pltpu.SemaphoreType.DMA] * 2,
            [pltpu.SemaphoreType.DMA] * 2,
        ],
    )
    def run(x_hbm, src_hbm, dst_hbm, out_hbm,
            src_v, dst_v, a_v, b_v, o_v, sem_i, sem_g, sem_o):
        wid = lax.axis_index("s") * _NC + lax.axis_index("c")
        base = wid * epw
        lanes = lax.iota(jnp.int32, _L)
        unroll = 8

        def issue_idx(c, p):
            cb = base + c * k
            return (
                pltpu.async_copy(src_hbm.at[pl.ds(cb, k)], src_v[p], sem_i[p]),
                pltpu.async_copy(dst_hbm.at[pl.ds(cb, k)], dst_v[p], sem_i[p]),
            )

        def issue_gather(p):
            return (
                pltpu.async_copy(x_hbm.at[src_v[p]], a_v[p], sem_g[p]),
                pltpu.async_copy(x_hbm.at[dst_v[p]], b_v[p], sem_g[p]),
            )

        def compute(c, p):
            def group_body(g, carry2):
                eids = g * _L + lanes

                def col_body(cb2, acc):
                    for u in range(unroll):
                        fv = (lanes + (cb2 * unroll + u)) & (d2 - 1)
                        apk = plsc.load_gather(a_v[p], [eids, fv])
                        bpk = plsc.load_gather(b_v[p], [eids, fv])
                        ab = plsc.bitcast(apk, jnp.bfloat16)
                        bb = plsc.bitcast(bpk, jnp.bfloat16)
                        plo, phi = plsc.unpack(
                            ab * bb, format=plsc.PackFormat.INTERLEAVED)
                        acc = acc + plo + phi
                    return acc

                acc = lax.fori_loop(0, d2 // unroll, col_body,
                                    jnp.zeros((_L,), jnp.float32))
                o_v[p][pl.ds(g * _L, _L)] = acc
                return carry2

            lax.fori_loop(0, groups, group_body, 0)
            return pltpu.async_copy(
                o_v[p], out_hbm.at[pl.ds(base + c * k, k)], sem_o[p])

        idx_cp = [None, None]
        gat_cp = [None, None]
        out_cp = [None, None]
        for cp in issue_idx(0, 0):
            cp.wait()
        gat_cp[0] = issue_gather(0)
        if n_chunks > 1:
            idx_cp[1] = issue_idx(1, 1)
        for c in range(n_chunks):
            p = c & 1
            for cp in gat_cp[p]:
                cp.wait()
            if c + 1 < n_chunks:
                for cp in idx_cp[p ^ 1]:
                    cp.wait()
                gat_cp[p ^ 1] = issue_gather(p ^ 1)
            if c + 2 < n_chunks:
                idx_cp[p] = issue_idx(c + 2, p)
            if out_cp[p] is not None:
                out_cp[p].wait()
            out_cp[p] = compute(c, p)
        for cp in out_cp:
            if cp is not None:
                cp.wait()

    return run(x_pk, src, dst)


def kernel(x, edge_index):
    n, d = x.shape
    n_edges = edge_index.shape[1]
    ei = edge_index.astype(jnp.int32)
    x_pk = lax.bitcast_convert_type(
        x.astype(jnp.bfloat16).reshape(n, d // 2, 2), jnp.int32)
    return _edge_dot_sc(x_pk, ei[0], ei[1], n_edges, d // 2)


# speedup vs baseline: 8.3015x; 1.0006x over previous
import functools

import jax
import jax.numpy as jnp
from jax import lax
from jax.experimental import pallas as pl
from jax.experimental.pallas import tpu as pltpu
from jax.experimental.pallas import tpu_sc as plsc

_INFO = plsc.get_sparse_core_info()
_NC = _INFO.num_cores
_NS = _INFO.num_subcores
_NW = _NC * _NS
_L = 16


def _edge_dot_sc(x_pk, src, dst, n_edges, d2):
    epw = n_edges // _NW
    k = 400 if epw % 400 == 0 else 16
    assert epw % k == 0 and k % _L == 0
    n_chunks = epw // k
    groups = k // _L

    mesh = plsc.VectorSubcoreMesh(core_axis_name="c", subcore_axis_name="s")

    @functools.partial(
        pl.kernel,
        mesh=mesh,
        out_type=jax.ShapeDtypeStruct((n_edges,), jnp.float32),
        compiler_params=pltpu.CompilerParams(
            needs_layout_passes=False, use_tc_tiling_on_sc=False,
            skip_device_barrier=True),
        scratch_types=[
            [pltpu.VMEM((k,), jnp.int32)] * 2,
            [pltpu.VMEM((k,), jnp.int32)] * 2,
            [pltpu.VMEM((k, d2), jnp.int32)] * 2,
            [pltpu.VMEM((k, d2), jnp.int32)] * 2,
            [pltpu.VMEM((k,), jnp.float32)] * 2,
            [pltpu.SemaphoreType.DMA] * 2,
            [pltpu.SemaphoreType.DMA] * 2,
            [pltpu.SemaphoreType.DMA] * 2,
        ],
    )
    def run(x_hbm, src_hbm, dst_hbm, out_hbm,
            src_v, dst_v, a_v, b_v, o_v, sem_i, sem_g, sem_o):
        wid = lax.axis_index("s") * _NC + lax.axis_index("c")
        base = wid * epw
        lanes = lax.iota(jnp.int32, _L)
        unroll = 8

        def issue_idx(c, p):
            cb = base + c * k
            return (
                pltpu.async_copy(src_hbm.at[pl.ds(cb, k)], src_v[p], sem_i[p]),
                pltpu.async_copy(dst_hbm.at[pl.ds(cb, k)], dst_v[p], sem_i[p]),
            )

        def issue_gather(p):
            return (
                pltpu.async_copy(x_hbm.at[src_v[p]], a_v[p], sem_g[p]),
                pltpu.async_copy(x_hbm.at[dst_v[p]], b_v[p], sem_g[p]),
            )

        def compute(c, p):
            def group_body(g, carry2):
                eids = g * _L + lanes

                def col_body(cb2, acc):
                    for u in range(unroll):
                        fv = (lanes + (cb2 * unroll + u)) & (d2 - 1)
                        apk = plsc.load_gather(a_v[p], [eids, fv])
                        bpk = plsc.load_gather(b_v[p], [eids, fv])
                        ab = plsc.bitcast(apk, jnp.bfloat16)
                        bb = plsc.bitcast(bpk, jnp.bfloat16)
                        plo, phi = plsc.unpack(
                            ab * bb, format=plsc.PackFormat.INTERLEAVED)
                        acc = acc + plo + phi
                    return acc

                acc = lax.fori_loop(0, d2 // unroll, col_body,
                                    jnp.zeros((_L,), jnp.float32))
                o_v[p][pl.ds(g * _L, _L)] = acc
                return carry2

            lax.fori_loop(0, groups, group_body, 0)
            return pltpu.async_copy(
                o_v[p], out_hbm.at[pl.ds(base + c * k, k)], sem_o[p])

        idx_cp = [None, None]
        gat_cp = [None, None]
        out_cp = [None, None]
        for cp in issue_idx(0, 0):
            cp.wait()
        gat_cp[0] = issue_gather(0)
        if n_chunks > 1:
            idx_cp[1] = issue_idx(1, 1)
        for c in range(n_chunks):
            p = c & 1
            for cp in gat_cp[p]:
                cp.wait()
            if c + 1 < n_chunks:
                for cp in idx_cp[p ^ 1]:
                    cp.wait()
                gat_cp[p ^ 1] = issue_gather(p ^ 1)
            if c + 2 < n_chunks:
                idx_cp[p] = issue_idx(c + 2, p)
            if out_cp[p] is not None:
                out_cp[p].wait()
            out_cp[p] = compute(c, p)
        for cp in out_cp:
            if cp is not None:
                cp.wait()

    return run(x_pk, src, dst)


def kernel(x, edge_index):
    n, d = x.shape
    n_edges = edge_index.shape[1]
    ei = edge_index.astype(jnp.int32)
    x_pk = lax.bitcast_convert_type(
        x.astype(jnp.bfloat16).reshape(n, d // 2, 2), jnp.int32)
    return _edge_dot_sc(x_pk, ei[0], ei[1], n_edges, d // 2)


# speedup vs baseline: 10.8935x; 1.3122x over previous
import functools

import jax
import jax.numpy as jnp
from jax import lax
from jax.experimental import pallas as pl
from jax.experimental.pallas import tpu as pltpu
from jax.experimental.pallas import tpu_sc as plsc

_INFO = plsc.get_sparse_core_info()
_NC = _INFO.num_cores
_NS = _INFO.num_subcores
_NW = _NC * _NS
_L = 16


def _edge_dot_sc(x_pk, ei, n_edges, d2):
    epw = n_edges // _NW
    k = 400 if epw % 400 == 0 else 16
    assert epw % k == 0 and k % _L == 0
    n_chunks = epw // k
    groups = k // _L
    n_blocks = d2 // _L

    mesh = plsc.VectorSubcoreMesh(core_axis_name="c", subcore_axis_name="s")

    @functools.partial(
        pl.kernel,
        mesh=mesh,
        out_type=jax.ShapeDtypeStruct((n_edges,), jnp.float32),
        compiler_params=pltpu.CompilerParams(
            needs_layout_passes=False, use_tc_tiling_on_sc=False),
        scratch_types=[
            [pltpu.VMEM((k,), jnp.int32)] * 2,
            [pltpu.VMEM((k,), jnp.int32)] * 2,
            [pltpu.VMEM((k, d2), jnp.int32)] * 2,
            [pltpu.VMEM((k, d2), jnp.int32)] * 2,
            [pltpu.VMEM((k,), jnp.float32)] * 2,
            [pltpu.SemaphoreType.DMA] * 2,
            [pltpu.SemaphoreType.DMA] * 2,
            [pltpu.SemaphoreType.DMA] * 2,
        ],
    )
    def run(x_hbm, ei_hbm, out_hbm,
            src_v, dst_v, a_v, b_v, o_v, sem_i, sem_g, sem_o):
        wid = lax.axis_index("s") * _NC + lax.axis_index("c")
        base = wid * epw
        lanes = lax.iota(jnp.int32, _L)
        rots = [(lanes + u) & (_L - 1) for u in range(_L)]

        def issue_idx(c, p):
            cb = base + c * k
            return (
                pltpu.async_copy(
                    ei_hbm.at[0, pl.ds(cb, k)], src_v[p], sem_i[p]),
                pltpu.async_copy(
                    ei_hbm.at[1, pl.ds(cb, k)], dst_v[p], sem_i[p]),
            )

        def issue_gather(p):
            return (
                pltpu.async_copy(x_hbm.at[src_v[p]], a_v[p], sem_g[p]),
                pltpu.async_copy(x_hbm.at[dst_v[p]], b_v[p], sem_g[p]),
            )

        def compute(c, p):
            def group_body(g, carry2):
                eids = g * _L + lanes

                def block_body(blk, accs):
                    acc0, acc1, acc2, acc3 = accs
                    blkv = jnp.full((_L,), blk * _L, jnp.int32)
                    for u in range(_L):
                        fv = blkv + rots[u]
                        apk = plsc.load_gather(a_v[p], [eids, fv])
                        bpk = plsc.load_gather(b_v[p], [eids, fv])
                        ab = plsc.bitcast(apk, jnp.bfloat16)
                        bb = plsc.bitcast(bpk, jnp.bfloat16)
                        plo, phi = plsc.unpack(
                            ab * bb, format=plsc.PackFormat.INTERLEAVED)
                        if u & 1:
                            acc2 = acc2 + plo
                            acc3 = acc3 + phi
                        else:
                            acc0 = acc0 + plo
                            acc1 = acc1 + phi
                    return acc0, acc1, acc2, acc3

                z = jnp.zeros((_L,), jnp.float32)
                acc0, acc1, acc2, acc3 = lax.fori_loop(
                    0, n_blocks, block_body, (z, z, z, z))
                o_v[p][pl.ds(g * _L, _L)] = (acc0 + acc1) + (acc2 + acc3)
                return carry2

            lax.fori_loop(0, groups, group_body, 0)
            return pltpu.async_copy(
                o_v[p], out_hbm.at[pl.ds(base + c * k, k)], sem_o[p])

        idx_cp = [None, None]
        gat_cp = [None, None]
        out_cp = [None, None]
        for cp in issue_idx(0, 0):
            cp.wait()
        gat_cp[0] = issue_gather(0)
        if n_chunks > 1:
            idx_cp[1] = issue_idx(1, 1)
        for c in range(n_chunks):
            p = c & 1
            for cp in gat_cp[p]:
                cp.wait()
            if c + 1 < n_chunks:
                for cp in idx_cp[p ^ 1]:
                    cp.wait()
                gat_cp[p ^ 1] = issue_gather(p ^ 1)
            if c + 2 < n_chunks:
                idx_cp[p] = issue_idx(c + 2, p)
            if out_cp[p] is not None:
                out_cp[p].wait()
            out_cp[p] = compute(c, p)
        for cp in out_cp:
            if cp is not None:
                cp.wait()

    return run(x_pk, ei)


def kernel(x, edge_index):
    n, d = x.shape
    n_edges = edge_index.shape[1]
    ei = edge_index.astype(jnp.int32)
    x_pk = lax.bitcast_convert_type(
        x.astype(jnp.bfloat16).reshape(n, d // 2, 2), jnp.int32)
    return _edge_dot_sc(x_pk, ei, n_edges, d // 2)


# speedup vs baseline: 11.9921x; 1.1008x over previous
import functools

import jax
import jax.numpy as jnp
from jax import lax
from jax.experimental import pallas as pl
from jax.experimental.pallas import tpu as pltpu
from jax.experimental.pallas import tpu_sc as plsc

_INFO = plsc.get_sparse_core_info()
_NC = _INFO.num_cores
_NS = _INFO.num_subcores
_NW = _NC * _NS
_L = 16


def _pack_sc(x, n, d):
    d2 = d // 2
    rows = 125
    n_items = n // rows
    assert n % rows == 0

    mesh = plsc.VectorSubcoreMesh(core_axis_name="c", subcore_axis_name="s")

    @functools.partial(
        pl.kernel,
        mesh=mesh,
        out_type=jax.ShapeDtypeStruct((n, d2), jnp.int32),
        compiler_params=pltpu.CompilerParams(
            needs_layout_passes=False, use_tc_tiling_on_sc=False),
        scratch_types=[
            pltpu.VMEM((rows, d), jnp.float32),
            pltpu.VMEM((rows, d2), jnp.int32),
        ],
    )
    def run(x_hbm, out_hbm, pv, qv):
        wid = lax.axis_index("s") * _NC + lax.axis_index("c")
        for j in range((n_items + _NW - 1) // _NW):
            item = wid + j * _NW

            @pl.when(item < n_items)
            def _():
                r0 = item * rows
                pltpu.sync_copy(x_hbm.at[pl.ds(r0, rows)], pv)

                def row_body(r, carry):
                    for q in range(d // 32):
                        v0 = pv[r, pl.ds(q * 32, _L)]
                        v1 = pv[r, pl.ds(q * 32 + _L, _L)]
                        pk = plsc.pack(
                            v0, v1, format=plsc.PackFormat.INTERLEAVED)
                        qv[r, pl.ds(q * _L, _L)] = plsc.bitcast(pk, jnp.int32)
                    return carry

                lax.fori_loop(0, rows, row_body, 0)
                pltpu.sync_copy(qv, out_hbm.at[pl.ds(r0, rows)])

    return run(x)


def _edge_dot_sc(x_pk, ei, n_edges, d2):
    epw = n_edges // _NW
    k = 400 if epw % 400 == 0 else 16
    assert epw % k == 0 and k % _L == 0
    n_chunks = epw // k
    groups = k // _L
    n_blocks = d2 // _L

    mesh = plsc.VectorSubcoreMesh(core_axis_name="c", subcore_axis_name="s")

    @functools.partial(
        pl.kernel,
        mesh=mesh,
        out_type=jax.ShapeDtypeStruct((n_edges,), jnp.float32),
        compiler_params=pltpu.CompilerParams(
            needs_layout_passes=False, use_tc_tiling_on_sc=False),
        scratch_types=[
            [pltpu.VMEM((k,), jnp.int32)] * 2,
            [pltpu.VMEM((k,), jnp.int32)] * 2,
            [pltpu.VMEM((k, d2), jnp.int32)] * 2,
            [pltpu.VMEM((k, d2), jnp.int32)] * 2,
            [pltpu.VMEM((k,), jnp.float32)] * 2,
            [pltpu.SemaphoreType.DMA] * 2,
            [pltpu.SemaphoreType.DMA] * 2,
            [pltpu.SemaphoreType.DMA] * 2,
        ],
    )
    def run(x_hbm, ei_hbm, out_hbm,
            src_v, dst_v, a_v, b_v, o_v, sem_i, sem_g, sem_o):
        wid = lax.axis_index("s") * _NC + lax.axis_index("c")
        base = wid * epw
        lanes = lax.iota(jnp.int32, _L)
        rots = [(lanes + u) & (_L - 1) for u in range(_L)]

        def issue_idx(c, p):
            cb = base + c * k
            return (
                pltpu.async_copy(
                    ei_hbm.at[0, pl.ds(cb, k)], src_v[p], sem_i[p]),
                pltpu.async_copy(
                    ei_hbm.at[1, pl.ds(cb, k)], dst_v[p], sem_i[p]),
            )

        def issue_gather(p):
            return (
                pltpu.async_copy(x_hbm.at[src_v[p]], a_v[p], sem_g[p]),
                pltpu.async_copy(x_hbm.at[dst_v[p]], b_v[p], sem_g[p]),
            )

        def compute(c, p):
            def group_body(g, carry2):
                eids = g * _L + lanes

                def block_body(blk, accs):
                    acc0, acc1, acc2, acc3 = accs
                    blkv = jnp.full((_L,), blk * _L, jnp.int32)
                    for u in range(_L):
                        fv = blkv + rots[u]
                        apk = plsc.load_gather(a_v[p], [eids, fv])
                        bpk = plsc.load_gather(b_v[p], [eids, fv])
                        ab = plsc.bitcast(apk, jnp.bfloat16)
                        bb = plsc.bitcast(bpk, jnp.bfloat16)
                        plo, phi = plsc.unpack(
                            ab * bb, format=plsc.PackFormat.INTERLEAVED)
                        if u & 1:
                            acc2 = acc2 + plo
                            acc3 = acc3 + phi
                        else:
                            acc0 = acc0 + plo
                            acc1 = acc1 + phi
                    return acc0, acc1, acc2, acc3

                z = jnp.zeros((_L,), jnp.float32)
                acc0, acc1, acc2, acc3 = lax.fori_loop(
                    0, n_blocks, block_body, (z, z, z, z))
                o_v[p][pl.ds(g * _L, _L)] = (acc0 + acc1) + (acc2 + acc3)
                return carry2

            lax.fori_loop(0, groups, group_body, 0)
            return pltpu.async_copy(
                o_v[p], out_hbm.at[pl.ds(base + c * k, k)], sem_o[p])

        idx_cp = [None, None]
        gat_cp = [None, None]
        out_cp = [None, None]
        for cp in issue_idx(0, 0):
            cp.wait()
        gat_cp[0] = issue_gather(0)
        if n_chunks > 1:
            idx_cp[1] = issue_idx(1, 1)
        for c in range(n_chunks):
            p = c & 1
            for cp in gat_cp[p]:
                cp.wait()
            if c + 1 < n_chunks:
                for cp in idx_cp[p ^ 1]:
                    cp.wait()
                gat_cp[p ^ 1] = issue_gather(p ^ 1)
            if c + 2 < n_chunks:
                idx_cp[p] = issue_idx(c + 2, p)
            if out_cp[p] is not None:
                out_cp[p].wait()
            out_cp[p] = compute(c, p)
        for cp in out_cp:
            if cp is not None:
                cp.wait()

    return run(x_pk, ei)


def kernel(x, edge_index):
    n, d = x.shape
    n_edges = edge_index.shape[1]
    ei = edge_index.astype(jnp.int32)
    x_pk = _pack_sc(x, n, d)
    return _edge_dot_sc(x_pk, ei, n_edges, d // 2)


# speedup vs baseline: 12.1156x; 1.0103x over previous
import functools

import jax
import jax.numpy as jnp
from jax import lax
from jax.experimental import pallas as pl
from jax.experimental.pallas import tpu as pltpu
from jax.experimental.pallas import tpu_sc as plsc

_INFO = plsc.get_sparse_core_info()
_NC = _INFO.num_cores
_NS = _INFO.num_subcores
_NW = _NC * _NS
_L = 16


def _edge_dot_sc(x, ei, n, d, n_edges):
    d2 = d // 2
    epw = n_edges // _NW
    k = 80 if epw % 80 == 0 else 16
    assert epw % k == 0 and k % _L == 0
    n_chunks = epw // k
    groups = k // _L
    n_blocks = d2 // _L
    rows_pt = n // _NS
    pchunk = 125
    p_trips = rows_pt // pchunk
    assert rows_pt % pchunk == 0

    mesh = plsc.VectorSubcoreMesh(core_axis_name="c", subcore_axis_name="s")

    @functools.partial(
        pl.kernel,
        mesh=mesh,
        out_type=jax.ShapeDtypeStruct((n_edges,), jnp.float32),
        compiler_params=pltpu.CompilerParams(
            needs_layout_passes=False, use_tc_tiling_on_sc=False),
        scratch_types=[
            pltpu.VMEM_SHARED((n, d2), jnp.int32),
            [pltpu.VMEM((pchunk, d), jnp.float32)] * 2,
            pltpu.VMEM((pchunk, d2), jnp.int32),
            pltpu.VMEM((epw,), jnp.int32),
            pltpu.VMEM((epw,), jnp.int32),
            [pltpu.VMEM((k, d2), jnp.int32)] * 2,
            [pltpu.VMEM((k, d2), jnp.int32)] * 2,
            [pltpu.VMEM((k,), jnp.float32)] * 2,
            [pltpu.SemaphoreType.DMA] * 2,
            pltpu.SemaphoreType.DMA,
            [pltpu.SemaphoreType.DMA] * 2,
            [pltpu.SemaphoreType.DMA] * 2,
        ],
    )
    def run(x_hbm, ei_hbm, out_hbm,
            xs, pv, qv, src_v, dst_v, a_v, b_v, o_v,
            sem_p, sem_i, sem_g, sem_o):
        cid = lax.axis_index("c")
        sid = lax.axis_index("s")
        wid = sid * _NC + cid
        base = wid * epw
        lanes = lax.iota(jnp.int32, _L)
        rots = [(lanes + u) & (_L - 1) for u in range(_L)]
        stripe = sid * rows_pt

        i_src = pltpu.async_copy(
            ei_hbm.at[0, pl.ds(base, epw)], src_v, sem_i)
        i_dst = pltpu.async_copy(
            ei_hbm.at[1, pl.ds(base, epw)], dst_v, sem_i)

        pk_cp = [pltpu.async_copy(
            x_hbm.at[pl.ds(stripe, pchunk)], pv[0], sem_p[0]), None]
        for t in range(p_trips):
            pt = t & 1
            pk_cp[pt].wait()
            if t + 1 < p_trips:
                pk_cp[pt ^ 1] = pltpu.async_copy(
                    x_hbm.at[pl.ds(stripe + (t + 1) * pchunk, pchunk)],
                    pv[pt ^ 1], sem_p[pt ^ 1])

            def row_body(r, carry):
                for q in range(d // 32):
                    v0 = pv[pt][r, pl.ds(q * 32, _L)]
                    v1 = pv[pt][r, pl.ds(q * 32 + _L, _L)]
                    pk = plsc.pack(v0, v1, format=plsc.PackFormat.INTERLEAVED)
                    qv[r, pl.ds(q * _L, _L)] = plsc.bitcast(pk, jnp.int32)
                return carry

            lax.fori_loop(0, pchunk, row_body, 0)
            pltpu.sync_copy(qv, xs.at[pl.ds(stripe + t * pchunk, pchunk)])
        plsc.subcore_barrier()
        i_src.wait()
        i_dst.wait()

        def issue_gather(c, p):
            return (
                pltpu.async_copy(
                    xs.at[src_v.at[pl.ds(c * k, k)]], a_v[p], sem_g[p]),
                pltpu.async_copy(
                    xs.at[dst_v.at[pl.ds(c * k, k)]], b_v[p], sem_g[p]),
            )

        def wait_gather(p):
            pltpu.make_async_copy(xs.at[src_v.at[pl.ds(0, k)]],
                                  a_v[p], sem_g[p]).wait()
            pltpu.make_async_copy(xs.at[dst_v.at[pl.ds(0, k)]],
                                  b_v[p], sem_g[p]).wait()

        def compute(c, p):
            def group_body(g, carry2):
                eids = g * _L + lanes

                def block_body(blk, accs):
                    acc0, acc1, acc2, acc3 = accs
                    blkv = jnp.full((_L,), blk * _L, jnp.int32)
                    for u in range(_L):
                        fv = blkv + rots[u]
                        apk = plsc.load_gather(a_v[p], [eids, fv])
                        bpk = plsc.load_gather(b_v[p], [eids, fv])
                        ab = plsc.bitcast(apk, jnp.bfloat16)
                        bb = plsc.bitcast(bpk, jnp.bfloat16)
                        plo, phi = plsc.unpack(
                            ab * bb, format=plsc.PackFormat.INTERLEAVED)
                        if u & 1:
                            acc2 = acc2 + plo
                            acc3 = acc3 + phi
                        else:
                            acc0 = acc0 + plo
                            acc1 = acc1 + phi
                    return acc0, acc1, acc2, acc3

                z = jnp.zeros((_L,), jnp.float32)
                acc0, acc1, acc2, acc3 = lax.fori_loop(
                    0, n_blocks, block_body, (z, z, z, z))
                o_v[p][pl.ds(g * _L, _L)] = (acc0 + acc1) + (acc2 + acc3)
                return carry2

            lax.fori_loop(0, groups, group_body, 0)
            pltpu.async_copy(
                o_v[p], out_hbm.at[pl.ds(base + c * k, k)], sem_o[p])

        def step(c, p):
            wait_gather(p)
            issue_gather(c + 1, p ^ 1)
            pltpu.make_async_copy(
                o_v[p], out_hbm.at[pl.ds(base, k)], sem_o[p]).wait()
            compute(c, p)

        issue_gather(0, 0)
        pltpu.async_copy(o_v[0], out_hbm.at[pl.ds(base, k)], sem_o[0])
        pltpu.async_copy(o_v[1], out_hbm.at[pl.ds(base + k, k)], sem_o[1])

        def pair_body(j, carry):
            step(2 * j, 0)
            step(2 * j + 1, 1)
            return carry

        lax.fori_loop(0, (n_chunks - 1) // 2, pair_body, 0)
        c_last = n_chunks - 1
        wait_gather(0)
        pltpu.make_async_copy(
            o_v[0], out_hbm.at[pl.ds(base, k)], sem_o[0]).wait()
        compute(c_last, 0)
        pltpu.make_async_copy(
            o_v[0], out_hbm.at[pl.ds(base, k)], sem_o[0]).wait()
        pltpu.make_async_copy(
            o_v[1], out_hbm.at[pl.ds(base, k)], sem_o[1]).wait()

    return run(x, ei)


def kernel(x, edge_index):
    n, d = x.shape
    n_edges = edge_index.shape[1]
    ei = edge_index.astype(jnp.int32)
    return _edge_dot_sc(x, ei, n, d, n_edges)
